# Initial kernel scaffold; baseline (speedup 1.0000x reference)
import jax
import jax.numpy as jnp
from jax.experimental import pallas as pl


def kernel(x, edge_index, W_l, b_l, W_r, b_r, att, bias):
    raise NotImplementedError("write your pallas kernel here")


# speedup vs baseline: 20.3833x; 20.3833x over previous
import functools

import jax
import jax.numpy as jnp
from jax import lax
from jax.experimental import pallas as pl
from jax.experimental.pallas import tpu as pltpu
from jax.experimental.pallas import tpu_sc as plsc

N = 10000
D = 256
H = 4
C = 64
HC = H * C
E = 160000
ETOT = E + N

NC = 2
NS = 16
L = 16

NPAD = 10240
B = 64
NB = 168
T = NB * B
EPAD = NS * T
ACCW = 144
RPT = NPAD // NS

MMR = 1280
FINR = 1000


def _mm_body(x_ref, wl_ref, bl_ref, wr_ref, br_ref, xl_ref, xr_ref):
    x = x_ref[...]
    xl_ref[...] = (
        jnp.dot(x, wl_ref[...], preferred_element_type=jnp.float32) + bl_ref[...]
    )
    xr_ref[...] = (
        jnp.dot(x, wr_ref[...], preferred_element_type=jnp.float32) + br_ref[...]
    )


_mm_call = pl.pallas_call(
    _mm_body,
    grid=(NC, NPAD // MMR),
    in_specs=[
        pl.BlockSpec((MMR, D), lambda p, r: (r, 0)),
        pl.BlockSpec((D, HC // NC), lambda p, r: (0, p)),
        pl.BlockSpec((None, 1, HC // NC), lambda p, r: (p, 0, 0)),
        pl.BlockSpec((D, HC // NC), lambda p, r: (0, p)),
        pl.BlockSpec((None, 1, HC // NC), lambda p, r: (p, 0, 0)),
    ],
    out_specs=[
        pl.BlockSpec((None, MMR, HC // NC), lambda p, r: (p, r, 0)),
        pl.BlockSpec((None, MMR, HC // NC), lambda p, r: (p, r, 0)),
    ],
    out_shape=[
        jax.ShapeDtypeStruct((NC, NPAD, HC // NC), jnp.float32),
        jax.ShapeDtypeStruct((NC, NPAD, HC // NC), jnp.float32),
    ],
)


def _lanesum(v):
    dnums = lax.GatherDimensionNumbers(
        offset_dims=(), collapsed_slice_dims=(0,), start_index_map=(0,)
    )
    for sh in (1, 2, 4, 8):
        idx = lax.iota(jnp.int32, L) ^ sh
        v = v + lax.gather(
            v,
            idx[:, None],
            dnums,
            (1,),
            mode=lax.GatherScatterMode.PROMISE_IN_BOUNDS,
        )
    return v


def _sc_body(xl_hbm, xr_hbm, src_hbm, dst_hbm, att_hbm, out_hbm,
             acc_sh, sidx, didx, glb, grb, xlb, xrb, msg, attv, sem):
    cid = lax.axis_index("c")
    sid = lax.axis_index("s")

    def _zrow(i, carry):
        for j in range(ACCW // L):
            msg[i, pl.ds(j * L, L)] = jnp.zeros((L,), jnp.float32)
        return carry

    lax.fori_loop(0, B, _zrow, 0)

    def _zacc(k, carry):
        pltpu.sync_copy(msg, acc_sh.at[pl.ds(sid * RPT + k * B, B)])
        return carry

    lax.fori_loop(0, RPT // B, _zacc, 0)
    plsc.subcore_barrier()

    pltpu.sync_copy(att_hbm.at[cid], attv)
    atv = [attv[h, pl.ds(j * L, L)] for h in range(2) for j in range(C // L)]

    off = cid * NPAD
    base = sid * T

    def _block(b, carry):
        e0 = base + b * B
        pltpu.sync_copy(src_hbm.at[pl.ds(e0, B)], sidx)
        pltpu.sync_copy(dst_hbm.at[pl.ds(e0, B)], didx)
        offv = jnp.full((L,), off, jnp.int32)
        for j in range(B // L):
            glb[pl.ds(j * L, L)] = sidx[pl.ds(j * L, L)] + offv
            grb[pl.ds(j * L, L)] = didx[pl.ds(j * L, L)] + offv
        cl = pltpu.async_copy(xl_hbm.at[glb], xlb, sem)
        cr = pltpu.async_copy(xr_hbm.at[grb], xrb, sem)
        cl.wait()
        cr.wait()

        def _edge(e, ecarry):
            xl = [xlb[e, pl.ds(j * L, L)] for j in range(8)]
            a0 = jnp.zeros((L,), jnp.float32)
            a1 = jnp.zeros((L,), jnp.float32)
            for j in range(8):
                u = xl[j] + xrb[e, pl.ds(j * L, L)]
                lr = jnp.where(u >= 0.0, u, 0.2 * u)
                if j < 4:
                    a0 = a0 + lr * atv[j]
                else:
                    a1 = a1 + lr * atv[j]
            p0 = jnp.exp(_lanesum(a0))
            p1 = jnp.exp(_lanesum(a1))
            for j in range(8):
                msg[e, pl.ds(j * L, L)] = (p0 if j < 4 else p1) * xl[j]
            lane = lax.iota(jnp.int32, L)
            dv = jnp.where(lane == 0, p0,
                           jnp.where(lane == 1, p1, jnp.zeros((L,), jnp.float32)))
            msg[e, pl.ds(8 * L, L)] = dv
            return ecarry

        lax.fori_loop(0, B, _edge, 0)
        pltpu.sync_copy(msg, acc_sh.at[didx], add=True)
        return carry

    lax.fori_loop(0, NB, _block, 0)
    plsc.subcore_barrier()
    pltpu.sync_copy(acc_sh.at[pl.ds(sid * RPT, RPT)],
                    out_hbm.at[cid, pl.ds(sid * RPT, RPT)])


_sc_call = pl.kernel(
    _sc_body,
    out_type=jax.ShapeDtypeStruct((NC, NPAD, ACCW), jnp.float32),
    mesh=plsc.VectorSubcoreMesh(core_axis_name="c", subcore_axis_name="s"),
    compiler_params=pltpu.CompilerParams(use_tc_tiling_on_sc=False),
    scratch_types=[
        pltpu.VMEM_SHARED((NPAD, ACCW), jnp.float32),
        pltpu.VMEM((B,), jnp.int32),
        pltpu.VMEM((B,), jnp.int32),
        pltpu.VMEM((B,), jnp.int32),
        pltpu.VMEM((B,), jnp.int32),
        pltpu.VMEM((B, HC // NC), jnp.float32),
        pltpu.VMEM((B, HC // NC), jnp.float32),
        pltpu.VMEM((B, ACCW), jnp.float32),
        pltpu.VMEM((2, C), jnp.float32),
        pltpu.SemaphoreType.DMA,
    ],
)


def _fin_body(acc_ref, b_ref, y_ref):
    a = acc_ref[...]
    m = a[:, : HC // NC]
    d0 = a[:, HC // NC : HC // NC + 1]
    d1 = a[:, HC // NC + 1 : HC // NC + 2]
    den = jnp.concatenate(
        [jnp.broadcast_to(d0, (FINR, C)), jnp.broadcast_to(d1, (FINR, C))], axis=1
    )
    v = m / den + b_ref[...]
    y_ref[...] = jnp.where(v > 0.0, v, jnp.exp(v) - 1.0)


_fin_call = pl.pallas_call(
    _fin_body,
    grid=(NC, N // FINR),
    in_specs=[
        pl.BlockSpec((None, FINR, ACCW), lambda p, r: (p, r, 0)),
        pl.BlockSpec((None, 1, HC // NC), lambda p, r: (p, 0, 0)),
    ],
    out_specs=pl.BlockSpec((FINR, HC // NC), lambda p, r: (r, p)),
    out_shape=jax.ShapeDtypeStruct((N, HC), jnp.float32),
)


def kernel(x, edge_index, W_l, b_l, W_r, b_r, att, bias):
    xp = jnp.pad(x, ((0, NPAD - N), (0, 0)))
    xl_tab, xr_tab = _mm_call(
        xp, W_l, b_l.reshape(NC, 1, HC // NC), W_r, b_r.reshape(NC, 1, HC // NC)
    )

    loop = jnp.arange(N, dtype=jnp.int32)
    pad = EPAD - ETOT
    src = jnp.concatenate(
        [edge_index[0], loop, jnp.zeros((pad,), jnp.int32)]
    )
    dst = jnp.concatenate(
        [edge_index[1], loop, jnp.full((pad,), N, jnp.int32)]
    )

    acc = _sc_call(
        xl_tab.reshape(NC * NPAD, HC // NC),
        xr_tab.reshape(NC * NPAD, HC // NC),
        src,
        dst,
        att.reshape(NC, 2, C),
    )
    return _fin_call(acc, bias.reshape(NC, 1, HC // NC))


# speedup vs baseline: 30.7536x; 1.5088x over previous
import jax
import jax.numpy as jnp
from jax import lax
from jax.experimental import pallas as pl
from jax.experimental.pallas import tpu as pltpu
from jax.experimental.pallas import tpu_sc as plsc

N = 10000
D = 256
H = 4
C = 64
HC = H * C
E = 160000
ETOT = E + N

NC = 2
NS = 16
L = 16

NPAD = 10240
B = 48
KB = 8
NSB = 28
NB = KB * NSB
T = NB * B
EPAD = NS * T
ACCW = 144
RPT = NPAD // NS

MMR = 1280
FINR = 1000


def _mm_body(x_ref, wl_ref, bl_ref, wr_ref, br_ref, xl_ref, xr_ref):
    x = x_ref[...]
    xl_ref[...] = (
        jnp.dot(x, wl_ref[...], preferred_element_type=jnp.float32) + bl_ref[...]
    )
    xr_ref[...] = (
        jnp.dot(x, wr_ref[...], preferred_element_type=jnp.float32) + br_ref[...]
    )


_mm_call = pl.pallas_call(
    _mm_body,
    grid=(NC, NPAD // MMR),
    in_specs=[
        pl.BlockSpec((MMR, D), lambda p, r: (r, 0)),
        pl.BlockSpec((D, HC // NC), lambda p, r: (0, p)),
        pl.BlockSpec((None, 1, HC // NC), lambda p, r: (p, 0, 0)),
        pl.BlockSpec((D, HC // NC), lambda p, r: (0, p)),
        pl.BlockSpec((None, 1, HC // NC), lambda p, r: (p, 0, 0)),
    ],
    out_specs=[
        pl.BlockSpec((None, MMR, HC // NC), lambda p, r: (p, r, 0)),
        pl.BlockSpec((None, MMR, HC // NC), lambda p, r: (p, r, 0)),
    ],
    out_shape=[
        jax.ShapeDtypeStruct((NC, NPAD, HC // NC), jnp.float32),
        jax.ShapeDtypeStruct((NC, NPAD, HC // NC), jnp.float32),
    ],
)


def _sc_body(xl_hbm, xr_hbm, src_hbm, dst_hbm, att_hbm, out_hbm,
             acc_sh, sidx2, didx2, glb, grb, xlb, xrb, msg, attv,
             sem0, sem1):
    cid = lax.axis_index("c")
    sid = lax.axis_index("s")
    sems = (sem0, sem1)

    def _zrow(i, carry):
        for j in range(ACCW // L):
            msg[i, pl.ds(j * L, L)] = jnp.zeros((L,), jnp.float32)
        return carry

    lax.fori_loop(0, B, _zrow, 0)

    def _zacc(k, carry):
        pltpu.sync_copy(msg, acc_sh.at[pl.ds(sid * RPT + k * B, B)])
        return carry

    lax.fori_loop(0, RPT // B, _zacc, 0)
    pltpu.sync_copy(msg.at[pl.ds(0, RPT - (RPT // B) * B)],
                    acc_sh.at[pl.ds(sid * RPT + (RPT // B) * B,
                                    RPT - (RPT // B) * B)])
    plsc.subcore_barrier()

    pltpu.sync_copy(att_hbm.at[cid], attv)
    atv = [attv[h, pl.ds(j * L, L)] for h in range(2) for j in range(C // L)]
    lane = lax.iota(jnp.int32, L)
    bfi = [lane ^ s for s in (1, 2, 4, 8)]
    m0 = lane == 0
    m1 = lane == 1
    zf = jnp.zeros((L,), jnp.float32)
    offv = jnp.full((L,), cid * NPAD, jnp.int32)
    dnums = lax.GatherDimensionNumbers(
        offset_dims=(), collapsed_slice_dims=(0,), start_index_map=(0,)
    )
    brow = sid * NB

    def _lanesum(v):
        for idx in bfi:
            v = v + lax.gather(
                v, idx[:, None], dnums, (1,),
                mode=lax.GatherScatterMode.PROMISE_IN_BOUNDS,
            )
        return v

    def _load_idx(sb, islot):
        pltpu.sync_copy(src_hbm.at[pl.ds(brow + sb * KB, KB)], sidx2.at[islot])
        pltpu.sync_copy(dst_hbm.at[pl.ds(brow + sb * KB, KB)], didx2.at[islot])

    def _issue(k, islot, gslot):
        for j in range(B // L):
            glb[gslot, pl.ds(j * L, L)] = sidx2[islot, k, pl.ds(j * L, L)] + offv
            grb[gslot, pl.ds(j * L, L)] = didx2[islot, k, pl.ds(j * L, L)] + offv
        pltpu.async_copy(xl_hbm.at[glb.at[gslot]], xlb.at[gslot], sems[gslot])
        pltpu.async_copy(xr_hbm.at[grb.at[gslot]], xrb.at[gslot], sems[gslot])

    def _edge_pair(i, carry, gslot):
        for e in (2 * i, 2 * i + 1):
            xl = [xlb[gslot, e, pl.ds(j * L, L)] for j in range(8)]
            a0 = zf
            a1 = zf
            for j in range(8):
                u = xl[j] + xrb[gslot, e, pl.ds(j * L, L)]
                t = u * atv[j]
                lt = jnp.where(u >= 0.0, t, 0.2 * t)
                if j < 4:
                    a0 = a0 + lt
                else:
                    a1 = a1 + lt
            p0 = jnp.exp(_lanesum(a0))
            p1 = jnp.exp(_lanesum(a1))
            for j in range(8):
                msg[e, pl.ds(j * L, L)] = (p0 if j < 4 else p1) * xl[j]
            msg[e, pl.ds(8 * L, L)] = jnp.where(m0, p0, jnp.where(m1, p1, zf))
        return carry

    def _compute(k, islot, gslot):
        pltpu.make_async_copy(xl_hbm.at[glb.at[gslot]], xlb.at[gslot],
                              sems[gslot]).wait()
        pltpu.make_async_copy(xr_hbm.at[grb.at[gslot]], xrb.at[gslot],
                              sems[gslot]).wait()
        lax.fori_loop(0, B // 2,
                      lambda i, c: _edge_pair(i, c, gslot), 0)
        pltpu.sync_copy(msg.at[pl.ds(0, B)], acc_sh.at[didx2.at[islot, k]],
                        add=True)

    _load_idx(0, 0)
    _issue(0, 0, 0)

    def _sb_pair(g2, carry):
        for half in range(2):
            sb = 2 * g2 + half

            @pl.when(sb + 1 < NSB)
            def _():
                _load_idx(sb + 1, 1 - half)

            for k in range(KB):
                gslot = k % 2
                if k < KB - 1:
                    _issue(k + 1, half, 1 - gslot)
                else:
                    @pl.when(sb + 1 < NSB)
                    def _():
                        _issue(0, 1 - half, 1 - gslot)
                _compute(k, half, gslot)
        return carry

    lax.fori_loop(0, NSB // 2, _sb_pair, 0)
    plsc.subcore_barrier()
    pltpu.sync_copy(acc_sh.at[pl.ds(sid * RPT, RPT)],
                    out_hbm.at[cid, pl.ds(sid * RPT, RPT)])


_sc_call = pl.kernel(
    _sc_body,
    out_type=jax.ShapeDtypeStruct((NC, NPAD, ACCW), jnp.float32),
    mesh=plsc.VectorSubcoreMesh(core_axis_name="c", subcore_axis_name="s"),
    compiler_params=pltpu.CompilerParams(use_tc_tiling_on_sc=False),
    scratch_types=[
        pltpu.VMEM_SHARED((NPAD, ACCW), jnp.float32),
        pltpu.VMEM((2, KB, B), jnp.int32),
        pltpu.VMEM((2, KB, B), jnp.int32),
        pltpu.VMEM((2, B), jnp.int32),
        pltpu.VMEM((2, B), jnp.int32),
        pltpu.VMEM((2, B, HC // NC), jnp.float32),
        pltpu.VMEM((2, B, HC // NC), jnp.float32),
        pltpu.VMEM((B, ACCW), jnp.float32),
        pltpu.VMEM((2, C), jnp.float32),
        pltpu.SemaphoreType.DMA,
        pltpu.SemaphoreType.DMA,
    ],
)


def _fin_body(acc_ref, b_ref, y_ref):
    a = acc_ref[...]
    m = a[:, : HC // NC]
    d0 = a[:, HC // NC : HC // NC + 1]
    d1 = a[:, HC // NC + 1 : HC // NC + 2]
    den = jnp.concatenate(
        [jnp.broadcast_to(d0, (FINR, C)), jnp.broadcast_to(d1, (FINR, C))], axis=1
    )
    v = m / den + b_ref[...]
    y_ref[...] = jnp.where(v > 0.0, v, jnp.exp(v) - 1.0)


_fin_call = pl.pallas_call(
    _fin_body,
    grid=(NC, N // FINR),
    in_specs=[
        pl.BlockSpec((None, FINR, ACCW), lambda p, r: (p, r, 0)),
        pl.BlockSpec((None, 1, HC // NC), lambda p, r: (p, 0, 0)),
    ],
    out_specs=pl.BlockSpec((FINR, HC // NC), lambda p, r: (r, p)),
    out_shape=jax.ShapeDtypeStruct((N, HC), jnp.float32),
)


def kernel(x, edge_index, W_l, b_l, W_r, b_r, att, bias):
    xp = jnp.pad(x, ((0, NPAD - N), (0, 0)))
    xl_tab, xr_tab = _mm_call(
        xp, W_l, b_l.reshape(NC, 1, HC // NC), W_r, b_r.reshape(NC, 1, HC // NC)
    )

    loop = jnp.arange(N, dtype=jnp.int32)
    pad = EPAD - ETOT
    src = jnp.concatenate([edge_index[0], loop, jnp.zeros((pad,), jnp.int32)])
    dst = jnp.concatenate([edge_index[1], loop, jnp.full((pad,), N, jnp.int32)])

    acc = _sc_call(
        xl_tab.reshape(NC * NPAD, HC // NC),
        xr_tab.reshape(NC * NPAD, HC // NC),
        src.reshape(EPAD // B, B),
        dst.reshape(EPAD // B, B),
        att.reshape(NC, 2, C),
    )
    return _fin_call(acc, bias.reshape(NC, 1, HC // NC))


# speedup vs baseline: 31.5572x; 1.0261x over previous
import jax
import jax.numpy as jnp
from jax import lax
from jax.experimental import pallas as pl
from jax.experimental.pallas import tpu as pltpu
from jax.experimental.pallas import tpu_sc as plsc

N = 10000
D = 256
H = 4
C = 64
HC = H * C
E = 160000
ETOT = E + N

NC = 2
NS = 16
L = 16

NPAD = 10240
B = 48
KB = 8
NSB = 28
NB = KB * NSB
T = NB * B
EPAD = NS * T
ACCW = 144
RPT = NPAD // NS

MMR = 1280
FINR = 1000


def _mm_body(x_ref, wl_ref, bl_ref, wr_ref, br_ref, xl_ref, xr_ref):
    x = x_ref[...]
    xl_ref[...] = (
        jnp.dot(x, wl_ref[...], preferred_element_type=jnp.float32) + bl_ref[...]
    )
    xr_ref[...] = (
        jnp.dot(x, wr_ref[...], preferred_element_type=jnp.float32) + br_ref[...]
    )


_mm_call = pl.pallas_call(
    _mm_body,
    grid=(NC, NPAD // MMR),
    in_specs=[
        pl.BlockSpec((MMR, D), lambda p, r: (r, 0)),
        pl.BlockSpec((D, HC // NC), lambda p, r: (0, p)),
        pl.BlockSpec((None, 1, HC // NC), lambda p, r: (p, 0, 0)),
        pl.BlockSpec((D, HC // NC), lambda p, r: (0, p)),
        pl.BlockSpec((None, 1, HC // NC), lambda p, r: (p, 0, 0)),
    ],
    out_specs=[
        pl.BlockSpec((None, MMR, HC // NC), lambda p, r: (p, r, 0)),
        pl.BlockSpec((None, MMR, HC // NC), lambda p, r: (p, r, 0)),
    ],
    out_shape=[
        jax.ShapeDtypeStruct((NC, NPAD, HC // NC), jnp.float32),
        jax.ShapeDtypeStruct((NC, NPAD, HC // NC), jnp.float32),
    ],
)


def _sc_body(xl_hbm, xr_hbm, src_hbm, dst_hbm, att_hbm, out_hbm,
             acc_sh, sidx2, didx2, glb, grb, xlb, xrb, msg, attv,
             sem0, sem1):
    cid = lax.axis_index("c")
    sid = lax.axis_index("s")
    sems = (sem0, sem1)

    def _zrow(i, carry):
        for j in range(ACCW // L):
            msg[i, pl.ds(j * L, L)] = jnp.zeros((L,), jnp.float32)
        return carry

    lax.fori_loop(0, B, _zrow, 0)

    def _zacc(k, carry):
        pltpu.sync_copy(msg, acc_sh.at[pl.ds(sid * RPT + k * B, B)])
        return carry

    lax.fori_loop(0, RPT // B, _zacc, 0)
    pltpu.sync_copy(msg.at[pl.ds(0, RPT - (RPT // B) * B)],
                    acc_sh.at[pl.ds(sid * RPT + (RPT // B) * B,
                                    RPT - (RPT // B) * B)])
    plsc.subcore_barrier()

    pltpu.sync_copy(att_hbm.at[cid], attv)
    atv = [attv[h, pl.ds(j * L, L)] for h in range(2) for j in range(C // L)]
    atv2 = [0.2 * a for a in atv]
    lane = lax.iota(jnp.int32, L)
    bfi = [lane ^ s for s in (1, 2, 4, 8)]
    m0 = lane == 0
    m1 = lane == 1
    zf = jnp.zeros((L,), jnp.float32)
    offv = jnp.full((L,), cid * NPAD, jnp.int32)
    dnums = lax.GatherDimensionNumbers(
        offset_dims=(), collapsed_slice_dims=(0,), start_index_map=(0,)
    )
    brow = sid * NB

    def _lanesum(v):
        for idx in bfi:
            v = v + lax.gather(
                v, idx[:, None], dnums, (1,),
                mode=lax.GatherScatterMode.PROMISE_IN_BOUNDS,
            )
        return v

    def _load_idx(sb, islot):
        pltpu.sync_copy(src_hbm.at[pl.ds(brow + sb * KB, KB)], sidx2.at[islot])
        pltpu.sync_copy(dst_hbm.at[pl.ds(brow + sb * KB, KB)], didx2.at[islot])

    def _issue(k, islot, gslot):
        for j in range(B // L):
            glb[gslot, pl.ds(j * L, L)] = sidx2[islot, k, pl.ds(j * L, L)] + offv
            grb[gslot, pl.ds(j * L, L)] = didx2[islot, k, pl.ds(j * L, L)] + offv
        pltpu.async_copy(xl_hbm.at[glb.at[gslot]], xlb.at[gslot], sems[gslot])
        pltpu.async_copy(xr_hbm.at[grb.at[gslot]], xrb.at[gslot], sems[gslot])

    def _edge_quad(i, carry, gslot):
        for e in (4 * i, 4 * i + 1, 4 * i + 2, 4 * i + 3):
            xl = [xlb[gslot, e, pl.ds(j * L, L)] for j in range(8)]
            acc = [zf, zf, zf, zf]
            for j in range(8):
                u = xl[j] + xrb[gslot, e, pl.ds(j * L, L)]
                lt = jnp.where(u >= 0.0, u * atv[j], u * atv2[j])
                k = (0 if j < 4 else 2) + (j & 1)
                acc[k] = acc[k] + lt
            p0 = jnp.exp(_lanesum(acc[0] + acc[1]))
            p1 = jnp.exp(_lanesum(acc[2] + acc[3]))
            for j in range(8):
                msg[e, pl.ds(j * L, L)] = (p0 if j < 4 else p1) * xl[j]
            msg[e, pl.ds(8 * L, L)] = jnp.where(m0, p0, jnp.where(m1, p1, zf))
        return carry

    def _compute(k, islot, gslot):
        pltpu.make_async_copy(xl_hbm.at[glb.at[gslot]], xlb.at[gslot],
                              sems[gslot]).wait()
        pltpu.make_async_copy(xr_hbm.at[grb.at[gslot]], xrb.at[gslot],
                              sems[gslot]).wait()
        lax.fori_loop(0, B // 4,
                      lambda i, c: _edge_quad(i, c, gslot), 0)
        pltpu.sync_copy(msg.at[pl.ds(0, B)], acc_sh.at[didx2.at[islot, k]],
                        add=True)

    _load_idx(0, 0)
    _issue(0, 0, 0)

    def _sb_pair(g2, carry):
        for half in range(2):
            sb = 2 * g2 + half

            @pl.when(sb + 1 < NSB)
            def _():
                _load_idx(sb + 1, 1 - half)

            for k in range(KB):
                gslot = k % 2
                if k < KB - 1:
                    _issue(k + 1, half, 1 - gslot)
                else:
                    @pl.when(sb + 1 < NSB)
                    def _():
                        _issue(0, 1 - half, 1 - gslot)
                _compute(k, half, gslot)
        return carry

    lax.fori_loop(0, NSB // 2, _sb_pair, 0)
    plsc.subcore_barrier()
    pltpu.sync_copy(acc_sh.at[pl.ds(sid * RPT, RPT)],
                    out_hbm.at[cid, pl.ds(sid * RPT, RPT)])


_sc_call = pl.kernel(
    _sc_body,
    out_type=jax.ShapeDtypeStruct((NC, NPAD, ACCW), jnp.float32),
    mesh=plsc.VectorSubcoreMesh(core_axis_name="c", subcore_axis_name="s"),
    compiler_params=pltpu.CompilerParams(use_tc_tiling_on_sc=False),
    scratch_types=[
        pltpu.VMEM_SHARED((NPAD, ACCW), jnp.float32),
        pltpu.VMEM((2, KB, B), jnp.int32),
        pltpu.VMEM((2, KB, B), jnp.int32),
        pltpu.VMEM((2, B), jnp.int32),
        pltpu.VMEM((2, B), jnp.int32),
        pltpu.VMEM((2, B, HC // NC), jnp.float32),
        pltpu.VMEM((2, B, HC // NC), jnp.float32),
        pltpu.VMEM((B, ACCW), jnp.float32),
        pltpu.VMEM((2, C), jnp.float32),
        pltpu.SemaphoreType.DMA,
        pltpu.SemaphoreType.DMA,
    ],
)


def _fin_body(acc_ref, b_ref, y_ref):
    a = acc_ref[...]
    m = a[:, : HC // NC]
    d0 = a[:, HC // NC : HC // NC + 1]
    d1 = a[:, HC // NC + 1 : HC // NC + 2]
    den = jnp.concatenate(
        [jnp.broadcast_to(d0, (FINR, C)), jnp.broadcast_to(d1, (FINR, C))], axis=1
    )
    v = m / den + b_ref[...]
    y_ref[...] = jnp.where(v > 0.0, v, jnp.exp(v) - 1.0)


_fin_call = pl.pallas_call(
    _fin_body,
    grid=(NC, N // FINR),
    in_specs=[
        pl.BlockSpec((None, FINR, ACCW), lambda p, r: (p, r, 0)),
        pl.BlockSpec((None, 1, HC // NC), lambda p, r: (p, 0, 0)),
    ],
    out_specs=pl.BlockSpec((FINR, HC // NC), lambda p, r: (r, p)),
    out_shape=jax.ShapeDtypeStruct((N, HC), jnp.float32),
)


def kernel(x, edge_index, W_l, b_l, W_r, b_r, att, bias):
    xp = jnp.pad(x, ((0, NPAD - N), (0, 0)))
    xl_tab, xr_tab = _mm_call(
        xp, W_l, b_l.reshape(NC, 1, HC // NC), W_r, b_r.reshape(NC, 1, HC // NC)
    )

    loop = jnp.arange(N, dtype=jnp.int32)
    pad = EPAD - ETOT
    src = jnp.concatenate([edge_index[0], loop, jnp.zeros((pad,), jnp.int32)])
    dst = jnp.concatenate([edge_index[1], loop, jnp.full((pad,), N, jnp.int32)])

    acc = _sc_call(
        xl_tab.reshape(NC * NPAD, HC // NC),
        xr_tab.reshape(NC * NPAD, HC // NC),
        src.reshape(EPAD // B, B),
        dst.reshape(EPAD // B, B),
        att.reshape(NC, 2, C),
    )
    return _fin_call(acc, bias.reshape(NC, 1, HC // NC))


# speedup vs baseline: 31.7541x; 1.0062x over previous
import jax
import jax.numpy as jnp
from jax import lax
from jax.experimental import pallas as pl
from jax.experimental.pallas import tpu as pltpu
from jax.experimental.pallas import tpu_sc as plsc

N = 10000
D = 256
H = 4
C = 64
HC = H * C
E = 160000
ETOT = E + N

NC = 2
NS = 16
L = 16

NPAD = 10240
B = 64
KB = 4
NSB = 42
NB = KB * NSB
T = NB * B
EPAD = NS * T
ACCW = 144
RPT = NPAD // NS

MMR = 1280
FINR = 1000


def _mm_body(x_ref, wl_ref, bl_ref, wr_ref, br_ref, xl_ref, xr_ref):
    x = x_ref[...]
    xl_ref[...] = (
        jnp.dot(x, wl_ref[...], preferred_element_type=jnp.float32) + bl_ref[...]
    )
    xr_ref[...] = (
        jnp.dot(x, wr_ref[...], preferred_element_type=jnp.float32) + br_ref[...]
    ).astype(jnp.bfloat16)


_mm_call = pl.pallas_call(
    _mm_body,
    grid=(NC, NPAD // MMR),
    in_specs=[
        pl.BlockSpec((MMR, D), lambda p, r: (r, 0)),
        pl.BlockSpec((D, HC // NC), lambda p, r: (0, p)),
        pl.BlockSpec((None, 1, HC // NC), lambda p, r: (p, 0, 0)),
        pl.BlockSpec((D, HC // NC), lambda p, r: (0, p)),
        pl.BlockSpec((None, 1, HC // NC), lambda p, r: (p, 0, 0)),
    ],
    out_specs=[
        pl.BlockSpec((None, MMR, HC // NC), lambda p, r: (p, r, 0)),
        pl.BlockSpec((None, MMR, HC // NC), lambda p, r: (p, r, 0)),
    ],
    out_shape=[
        jax.ShapeDtypeStruct((NC, NPAD, HC // NC), jnp.float32),
        jax.ShapeDtypeStruct((NC, NPAD, HC // NC), jnp.bfloat16),
    ],
)


def _sc_body(xl_hbm, xr_hbm, src_hbm, dst_hbm, att_hbm, out_hbm,
             acc_sh, sidx2, didx2, glb, grb, xlb, xrb, msg, attv,
             sem0, sem1):
    cid = lax.axis_index("c")
    sid = lax.axis_index("s")
    sems = (sem0, sem1)

    def _zrow(i, carry):
        for j in range(ACCW // L):
            msg[i, pl.ds(j * L, L)] = jnp.zeros((L,), jnp.float32)
        return carry

    lax.fori_loop(0, B, _zrow, 0)

    def _zacc(k, carry):
        pltpu.sync_copy(msg, acc_sh.at[pl.ds(sid * RPT + k * B, B)])
        return carry

    lax.fori_loop(0, RPT // B, _zacc, 0)
    plsc.subcore_barrier()

    pltpu.sync_copy(att_hbm.at[cid], attv)
    atv = [attv[h, pl.ds(j * L, L)] for h in range(2) for j in range(C // L)]
    atv2 = [0.2 * a for a in atv]
    lane = lax.iota(jnp.int32, L)
    bfi = [lane ^ s for s in (1, 2, 4, 8)]
    m0 = lane == 0
    m1 = lane == 1
    zf = jnp.zeros((L,), jnp.float32)
    offv = jnp.full((L,), cid * NPAD, jnp.int32)
    dnums = lax.GatherDimensionNumbers(
        offset_dims=(), collapsed_slice_dims=(0,), start_index_map=(0,)
    )
    brow = sid * NB

    def _lanesum(v):
        for idx in bfi:
            v = v + lax.gather(
                v, idx[:, None], dnums, (1,),
                mode=lax.GatherScatterMode.PROMISE_IN_BOUNDS,
            )
        return v

    def _load_idx(sb, islot):
        pltpu.sync_copy(src_hbm.at[pl.ds(brow + sb * KB, KB)], sidx2.at[islot])
        pltpu.sync_copy(dst_hbm.at[pl.ds(brow + sb * KB, KB)], didx2.at[islot])

    def _issue(k, islot, gslot):
        for j in range(B // L):
            glb[gslot, pl.ds(j * L, L)] = sidx2[islot, k, pl.ds(j * L, L)] + offv
            grb[gslot, pl.ds(j * L, L)] = didx2[islot, k, pl.ds(j * L, L)] + offv
        pltpu.async_copy(xl_hbm.at[glb.at[gslot]], xlb.at[gslot], sems[gslot])
        pltpu.async_copy(xr_hbm.at[grb.at[gslot]], xrb.at[gslot], sems[gslot])

    def _edge_quad(i, carry, gslot):
        for e in (4 * i, 4 * i + 1, 4 * i + 2, 4 * i + 3):
            xl = [xlb[gslot, e, pl.ds(j * L, L)] for j in range(8)]
            xr = []
            for j2 in range(4):
                ev, od = plsc.unpack(
                    xrb[gslot, e, pl.ds(j2 * 2 * L, 2 * L)],
                    format=plsc.PackFormat.INTERLEAVED,
                )
                xr.append(ev)
                xr.append(od)
            acc = [zf, zf, zf, zf]
            for j in range(8):
                u = xl[j] + xr[j]
                lt = jnp.where(u >= 0.0, u * atv[j], u * atv2[j])
                k = (0 if j < 4 else 2) + (j & 1)
                acc[k] = acc[k] + lt
            p0 = jnp.exp(_lanesum(acc[0] + acc[1]))
            p1 = jnp.exp(_lanesum(acc[2] + acc[3]))
            for j in range(8):
                msg[e, pl.ds(j * L, L)] = (p0 if j < 4 else p1) * xl[j]
            msg[e, pl.ds(8 * L, L)] = jnp.where(m0, p0, jnp.where(m1, p1, zf))
        return carry

    def _compute(k, islot, gslot):
        pltpu.make_async_copy(xl_hbm.at[glb.at[gslot]], xlb.at[gslot],
                              sems[gslot]).wait()
        pltpu.make_async_copy(xr_hbm.at[grb.at[gslot]], xrb.at[gslot],
                              sems[gslot]).wait()
        lax.fori_loop(0, B // 4,
                      lambda i, c: _edge_quad(i, c, gslot), 0)
        pltpu.sync_copy(msg.at[pl.ds(0, B)], acc_sh.at[didx2.at[islot, k]],
                        add=True)

    _load_idx(0, 0)
    _issue(0, 0, 0)

    def _sb_pair(g2, carry):
        for half in range(2):
            sb = 2 * g2 + half

            @pl.when(sb + 1 < NSB)
            def _():
                _load_idx(sb + 1, 1 - half)

            for k in range(KB):
                gslot = k % 2
                if k < KB - 1:
                    _issue(k + 1, half, 1 - gslot)
                else:
                    @pl.when(sb + 1 < NSB)
                    def _():
                        _issue(0, 1 - half, 1 - gslot)
                _compute(k, half, gslot)
        return carry

    lax.fori_loop(0, NSB // 2, _sb_pair, 0)
    plsc.subcore_barrier()
    pltpu.sync_copy(acc_sh.at[pl.ds(sid * RPT, RPT)],
                    out_hbm.at[cid, pl.ds(sid * RPT, RPT)])


_sc_call = pl.kernel(
    _sc_body,
    out_type=jax.ShapeDtypeStruct((NC, NPAD, ACCW), jnp.float32),
    mesh=plsc.VectorSubcoreMesh(core_axis_name="c", subcore_axis_name="s"),
    compiler_params=pltpu.CompilerParams(use_tc_tiling_on_sc=False,
                                         needs_layout_passes=False),
    scratch_types=[
        pltpu.VMEM_SHARED((NPAD, ACCW), jnp.float32),
        pltpu.VMEM((2, KB, B), jnp.int32),
        pltpu.VMEM((2, KB, B), jnp.int32),
        pltpu.VMEM((2, B), jnp.int32),
        pltpu.VMEM((2, B), jnp.int32),
        pltpu.VMEM((2, B, HC // NC), jnp.float32),
        pltpu.VMEM((2, B, HC // NC), jnp.bfloat16),
        pltpu.VMEM((B, ACCW), jnp.float32),
        pltpu.VMEM((2, C), jnp.float32),
        pltpu.SemaphoreType.DMA,
        pltpu.SemaphoreType.DMA,
    ],
)


def _fin_body(acc_ref, b_ref, y_ref):
    a = acc_ref[...]
    m = a[:, : HC // NC]
    d0 = a[:, HC // NC : HC // NC + 1]
    d1 = a[:, HC // NC + 1 : HC // NC + 2]
    den = jnp.concatenate(
        [jnp.broadcast_to(d0, (FINR, C)), jnp.broadcast_to(d1, (FINR, C))], axis=1
    )
    v = m / den + b_ref[...]
    y_ref[...] = jnp.where(v > 0.0, v, jnp.exp(v) - 1.0)


_fin_call = pl.pallas_call(
    _fin_body,
    grid=(NC, N // FINR),
    in_specs=[
        pl.BlockSpec((None, FINR, ACCW), lambda p, r: (p, r, 0)),
        pl.BlockSpec((None, 1, HC // NC), lambda p, r: (p, 0, 0)),
    ],
    out_specs=pl.BlockSpec((FINR, HC // NC), lambda p, r: (r, p)),
    out_shape=jax.ShapeDtypeStruct((N, HC), jnp.float32),
)


_PERM = []
for _p in range(NC):
    for _j2 in range(4):
        _base = _p * 128 + 32 * _j2
        for _t in range(16):
            _PERM.append(_base + _t)
            _PERM.append(_base + 16 + _t)
_PERM = tuple(_PERM)


def kernel(x, edge_index, W_l, b_l, W_r, b_r, att, bias):
    xp = jnp.pad(x, ((0, NPAD - N), (0, 0)))
    perm = jnp.array(_PERM, dtype=jnp.int32)
    W_rp = W_r[:, perm]
    b_rp = b_r[perm]
    xl_tab, xr_tab = _mm_call(
        xp, W_l, b_l.reshape(NC, 1, HC // NC), W_rp, b_rp.reshape(NC, 1, HC // NC)
    )

    loop = jnp.arange(N, dtype=jnp.int32)
    pad = EPAD - ETOT
    src = jnp.concatenate([edge_index[0], loop, jnp.zeros((pad,), jnp.int32)])
    dst = jnp.concatenate([edge_index[1], loop, jnp.full((pad,), N, jnp.int32)])

    acc = _sc_call(
        xl_tab.reshape(NC * NPAD, HC // NC),
        xr_tab.reshape(NC * NPAD, HC // NC),
        src.reshape(EPAD // B, B),
        dst.reshape(EPAD // B, B),
        att.reshape(NC, 2, C),
    )
    return _fin_call(acc, bias.reshape(NC, 1, HC // NC))


# speedup vs baseline: 43.0333x; 1.3552x over previous
import jax
import jax.numpy as jnp
from jax import lax
from jax.experimental import pallas as pl
from jax.experimental.pallas import tpu as pltpu
from jax.experimental.pallas import tpu_sc as plsc

N = 10000
D = 256
H = 4
C = 64
HC = H * C
E = 160000
ETOT = E + N

NC = 2
NS = 16
L = 16

NPAD = 10240
B = 64
KB = 4
NSB = 42
NB = KB * NSB
T = NB * B
EPAD = NS * T
ACCW = 144
RPT = NPAD // NS

MMR = 1280
FINR = 1000


def _mm_body(x_ref, wl_ref, bl_ref, wr_ref, br_ref, xl_ref, xr_ref):
    x = x_ref[...]
    xl_ref[...] = (
        jnp.dot(x, wl_ref[...], preferred_element_type=jnp.float32) + bl_ref[...]
    ).astype(jnp.bfloat16)
    xr_ref[...] = (
        jnp.dot(x, wr_ref[...], preferred_element_type=jnp.float32) + br_ref[...]
    ).astype(jnp.bfloat16)


_mm_call = pl.pallas_call(
    _mm_body,
    grid=(NC, NPAD // MMR),
    in_specs=[
        pl.BlockSpec((MMR, D), lambda p, r: (r, 0)),
        pl.BlockSpec((D, HC // NC), lambda p, r: (0, p)),
        pl.BlockSpec((None, 1, HC // NC), lambda p, r: (p, 0, 0)),
        pl.BlockSpec((D, HC // NC), lambda p, r: (0, p)),
        pl.BlockSpec((None, 1, HC // NC), lambda p, r: (p, 0, 0)),
    ],
    out_specs=[
        pl.BlockSpec((None, MMR, HC // NC), lambda p, r: (p, r, 0)),
        pl.BlockSpec((None, MMR, HC // NC), lambda p, r: (p, r, 0)),
    ],
    out_shape=[
        jax.ShapeDtypeStruct((NC, NPAD, HC // NC), jnp.bfloat16),
        jax.ShapeDtypeStruct((NC, NPAD, HC // NC), jnp.bfloat16),
    ],
)


def _sc_body(xl_hbm, xr_hbm, src_hbm, dst_hbm, att_hbm, out_hbm,
             acc_sh, sidx2, didx2, glb, grb, xlb, xrb, msg, attv,
             sem0, sem1, ssem0, ssem1):
    cid = lax.axis_index("c")
    sid = lax.axis_index("s")
    sems = (sem0, sem1)
    ssems = (ssem0, ssem1)

    def _zrow(i, carry):
        for j in range(ACCW // L):
            msg[0, i, pl.ds(j * L, L)] = jnp.zeros((L,), jnp.float32)
        return carry

    lax.fori_loop(0, B, _zrow, 0)

    def _zacc(k, carry):
        pltpu.sync_copy(msg.at[0], acc_sh.at[pl.ds(sid * RPT + k * B, B)])
        return carry

    lax.fori_loop(0, RPT // B, _zacc, 0)
    plsc.subcore_barrier()

    pltpu.sync_copy(att_hbm.at[cid], attv)
    atv = [attv[h, pl.ds(j * L, L)] for h in range(2) for j in range(C // L)]
    atv2 = [0.2 * a for a in atv]
    lane = lax.iota(jnp.int32, L)
    bfi = [lane ^ s for s in (1, 2, 4, 8)]
    m0 = lane == 0
    m1 = lane == 1
    zf = jnp.zeros((L,), jnp.float32)
    offv = jnp.full((L,), cid * NPAD, jnp.int32)
    dnums = lax.GatherDimensionNumbers(
        offset_dims=(), collapsed_slice_dims=(0,), start_index_map=(0,)
    )
    i15 = jnp.full((L,), L - 1, jnp.int32)
    brow = sid * NB

    def _bcast15(v):
        return lax.gather(
            v, i15[:, None], dnums, (1,),
            mode=lax.GatherScatterMode.PROMISE_IN_BOUNDS,
        )

    def _load_idx(sb, islot):
        pltpu.sync_copy(src_hbm.at[pl.ds(brow + sb * KB, KB)], sidx2.at[islot])
        pltpu.sync_copy(dst_hbm.at[pl.ds(brow + sb * KB, KB)], didx2.at[islot])

    def _issue(k, islot, gslot):
        for j in range(B // L):
            glb[gslot, pl.ds(j * L, L)] = sidx2[islot, k, pl.ds(j * L, L)] + offv
            grb[gslot, pl.ds(j * L, L)] = didx2[islot, k, pl.ds(j * L, L)] + offv
        pltpu.async_copy(xl_hbm.at[glb.at[gslot]], xlb.at[gslot], sems[gslot])
        pltpu.async_copy(xr_hbm.at[grb.at[gslot]], xrb.at[gslot], sems[gslot])

    def _edge_pair2(i, carry, gslot):
        es = (2 * i, 2 * i + 1)
        xls = []
        sums = []
        for e in es:
            xl = []
            xr = []
            for j2 in range(4):
                lev, lod = plsc.unpack(
                    xlb[gslot, e, pl.ds(j2 * 2 * L, 2 * L)],
                    format=plsc.PackFormat.INTERLEAVED,
                )
                xl.append(lev)
                xl.append(lod)
                rev, rod = plsc.unpack(
                    xrb[gslot, e, pl.ds(j2 * 2 * L, 2 * L)],
                    format=plsc.PackFormat.INTERLEAVED,
                )
                xr.append(rev)
                xr.append(rod)
            acc = [None, None, None, None]
            for j in range(8):
                u = xl[j] + xr[j]
                lt = jnp.where(u >= 0.0, u * atv[j], u * atv2[j])
                k = (0 if j < 4 else 2) + (j & 1)
                acc[k] = lt if acc[k] is None else acc[k] + lt
            xls.append(xl)
            sums.append((acc[0] + acc[1], acc[2] + acc[3]))
        cums = [plsc.cumsum(sums[ei][h]) for ei in range(2) for h in range(2)]
        ps = [jnp.exp(_bcast15(c)) for c in cums]
        for ei, e in enumerate(es):
            p0 = ps[2 * ei]
            p1 = ps[2 * ei + 1]
            for j in range(8):
                msg[gslot, e, pl.ds(j * L, L)] = (p0 if j < 4 else p1) * xls[ei][j]
            msg[gslot, e, pl.ds(8 * L, L)] = jnp.where(
                m0, p0, jnp.where(m1, p1, zf))
        return carry

    def _scat_wait(k, islot, gslot):
        pltpu.make_async_copy(msg.at[gslot], acc_sh.at[didx2.at[islot, k]],
                              ssems[gslot]).wait()

    def _compute(sb, k, islot, gslot):
        pltpu.make_async_copy(xl_hbm.at[glb.at[gslot]], xlb.at[gslot],
                              sems[gslot]).wait()
        pltpu.make_async_copy(xr_hbm.at[grb.at[gslot]], xrb.at[gslot],
                              sems[gslot]).wait()
        if k < 2:
            @pl.when(sb > 0)
            def _():
                _scat_wait(k, islot, gslot)
        else:
            _scat_wait(k, islot, gslot)
        lax.fori_loop(0, B // 2,
                      lambda i, c: _edge_pair2(i, c, gslot), 0)
        pltpu.async_copy(msg.at[gslot], acc_sh.at[didx2.at[islot, k]],
                         ssems[gslot], add=True)

    _load_idx(0, 0)
    _issue(0, 0, 0)

    def _sb_pair(g2, carry):
        for half in range(2):
            sb = 2 * g2 + half

            @pl.when(sb + 1 < NSB)
            def _():
                _load_idx(sb + 1, 1 - half)

            for k in range(KB):
                gslot = k % 2
                if k < KB - 1:
                    _issue(k + 1, half, 1 - gslot)
                else:
                    @pl.when(sb + 1 < NSB)
                    def _():
                        _issue(0, 1 - half, 1 - gslot)
                _compute(sb, k, half, gslot)
        return carry

    lax.fori_loop(0, NSB // 2, _sb_pair, 0)
    _scat_wait(KB - 2, 1, 0)
    _scat_wait(KB - 1, 1, 1)
    plsc.subcore_barrier()
    pltpu.sync_copy(acc_sh.at[pl.ds(sid * RPT, RPT)],
                    out_hbm.at[cid, pl.ds(sid * RPT, RPT)])


_sc_call = pl.kernel(
    _sc_body,
    out_type=jax.ShapeDtypeStruct((NC, NPAD, ACCW), jnp.float32),
    mesh=plsc.VectorSubcoreMesh(core_axis_name="c", subcore_axis_name="s"),
    compiler_params=pltpu.CompilerParams(use_tc_tiling_on_sc=False,
                                         needs_layout_passes=False),
    scratch_types=[
        pltpu.VMEM_SHARED((NPAD, ACCW), jnp.float32),
        pltpu.VMEM((2, KB, B), jnp.int32),
        pltpu.VMEM((2, KB, B), jnp.int32),
        pltpu.VMEM((2, B), jnp.int32),
        pltpu.VMEM((2, B), jnp.int32),
        pltpu.VMEM((2, B, HC // NC), jnp.bfloat16),
        pltpu.VMEM((2, B, HC // NC), jnp.bfloat16),
        pltpu.VMEM((2, B, ACCW), jnp.float32),
        pltpu.VMEM((2, C), jnp.float32),
        pltpu.SemaphoreType.DMA,
        pltpu.SemaphoreType.DMA,
        pltpu.SemaphoreType.DMA,
        pltpu.SemaphoreType.DMA,
    ],
)


def _fin_body(acc_ref, b_ref, y_ref):
    a = acc_ref[...]
    m = a[:, : HC // NC]
    d0 = a[:, HC // NC : HC // NC + 1]
    d1 = a[:, HC // NC + 1 : HC // NC + 2]
    den = jnp.concatenate(
        [jnp.broadcast_to(d0, (FINR, C)), jnp.broadcast_to(d1, (FINR, C))], axis=1
    )
    v = m / den + b_ref[...]
    y_ref[...] = jnp.where(v > 0.0, v, jnp.exp(v) - 1.0)


_fin_call = pl.pallas_call(
    _fin_body,
    grid=(NC, N // FINR),
    in_specs=[
        pl.BlockSpec((None, FINR, ACCW), lambda p, r: (p, r, 0)),
        pl.BlockSpec((None, 1, HC // NC), lambda p, r: (p, 0, 0)),
    ],
    out_specs=pl.BlockSpec((FINR, HC // NC), lambda p, r: (r, p)),
    out_shape=jax.ShapeDtypeStruct((N, HC), jnp.float32),
)


_PERM = []
for _p in range(NC):
    for _j2 in range(4):
        _base = _p * 128 + 32 * _j2
        for _t in range(16):
            _PERM.append(_base + _t)
            _PERM.append(_base + 16 + _t)
_PERM = tuple(_PERM)


def kernel(x, edge_index, W_l, b_l, W_r, b_r, att, bias):
    xp = jnp.pad(x, ((0, NPAD - N), (0, 0)))
    perm = jnp.array(_PERM, dtype=jnp.int32)
    W_lp = W_l[:, perm]
    b_lp = b_l[perm]
    W_rp = W_r[:, perm]
    b_rp = b_r[perm]
    xl_tab, xr_tab = _mm_call(
        xp, W_lp, b_lp.reshape(NC, 1, HC // NC), W_rp,
        b_rp.reshape(NC, 1, HC // NC)
    )

    loop = jnp.arange(N, dtype=jnp.int32)
    pad = EPAD - ETOT
    src = jnp.concatenate([edge_index[0], loop, jnp.zeros((pad,), jnp.int32)])
    dst = jnp.concatenate([edge_index[1], loop, jnp.full((pad,), N, jnp.int32)])

    acc = _sc_call(
        xl_tab.reshape(NC * NPAD, HC // NC),
        xr_tab.reshape(NC * NPAD, HC // NC),
        src.reshape(EPAD // B, B),
        dst.reshape(EPAD // B, B),
        att.reshape(NC, 2, C),
    )
    return _fin_call(acc, bias.reshape(NC, 1, HC // NC))
